# Initial kernel scaffold; baseline (speedup 1.0000x reference)
import jax
import jax.numpy as jnp
from jax.experimental import pallas as pl


def kernel(xyz, points, W0, b0, gamma0, beta0, W1, b1, gamma1, beta1):
    raise NotImplementedError("write your pallas kernel here")


# speedup vs baseline: 1.3568x; 1.3568x over previous
import jax
import jax.numpy as jnp
from jax.experimental import pallas as pl
from jax.experimental.pallas import tpu as pltpu

_EPS = 1e-5
_BLK = 8192


def _body(nb_total, cnt, nbatch,
          xyz_ref, pts_ref, w0x_ref, w0p_ref, b0_ref, g0_ref, be0_ref,
          w1_ref, b1_ref, g1_ref, be1_ref, out_ref,
          s0_ref, q0_ref, s1_ref, q1_ref, mx_ref, mn_ref):
    ph = pl.program_id(0)
    b = pl.program_id(1)
    nb = pl.program_id(2)

    @pl.when((ph == 0) & (b == 0) & (nb == 0))
    def _init():
        s0_ref[:] = jnp.zeros_like(s0_ref)
        q0_ref[:] = jnp.zeros_like(q0_ref)
        s1_ref[:] = jnp.zeros_like(s1_ref)
        q1_ref[:] = jnp.zeros_like(q1_ref)
        mx_ref[:] = jnp.full_like(mx_ref, -jnp.inf)
        mn_ref[:] = jnp.full_like(mn_ref, jnp.inf)

    x = xyz_ref[0]
    p = pts_ref[0]
    y0 = (jax.lax.dot_general(w0x_ref[:], x, (((1,), (0,)), ((), ())),
                              preferred_element_type=jnp.float32)
          + jax.lax.dot_general(w0p_ref[:], p, (((1,), (0,)), ((), ())),
                                preferred_element_type=jnp.float32)
          + b0_ref[:])

    @pl.when(ph == 0)
    def _pass0():
        s0_ref[:] += jnp.sum(y0, axis=1, keepdims=True)
        q0_ref[:] += jnp.sum(y0 * y0, axis=1, keepdims=True)

    @pl.when(ph == 1)
    def _pass1():
        m0 = s0_ref[:] / cnt
        v0 = q0_ref[:] / cnt - m0 * m0
        a0 = g0_ref[:] * jax.lax.rsqrt(v0 + _EPS)
        c0 = be0_ref[:] - a0 * m0
        act0 = jnp.maximum(a0 * y0 + c0, 0.0)
        h1 = jax.lax.dot_general(w1_ref[:], act0, (((1,), (0,)), ((), ())),
                                 preferred_element_type=jnp.float32) + b1_ref[:]
        s1_ref[:] += jnp.sum(h1, axis=1, keepdims=True)
        q1_ref[:] += jnp.sum(h1 * h1, axis=1, keepdims=True)
        bmax = jnp.max(h1, axis=1, keepdims=True)
        bmin = jnp.min(h1, axis=1, keepdims=True)
        lane = jax.lax.broadcasted_iota(jnp.int32, mx_ref.shape, 1)
        hit = lane == b
        mx_ref[:] = jnp.where(hit, jnp.maximum(mx_ref[:], bmax), mx_ref[:])
        mn_ref[:] = jnp.where(hit, jnp.minimum(mn_ref[:], bmin), mn_ref[:])

    @pl.when((ph == 1) & (b == nbatch - 1) & (nb == nb_total - 1))
    def _final():
        m1 = s1_ref[:] / cnt
        v1 = q1_ref[:] / cnt - m1 * m1
        a1 = g1_ref[:] * jax.lax.rsqrt(v1 + _EPS)
        c1 = be1_ref[:] - a1 * m1
        pick = jnp.where(a1 >= 0.0, mx_ref[:], mn_ref[:])
        out_ref[:] = jnp.maximum(a1 * pick + c1, 0.0)


def kernel(xyz, points, W0, b0, gamma0, beta0, W1, b1, gamma1, beta1):
    B, _, N = xyz.shape
    D = points.shape[1]
    C0 = W0.shape[0]
    C1 = W1.shape[0]
    blk = _BLK if N % _BLK == 0 else N
    nb_total = N // blk
    cnt = float(B * N)

    col = lambda v: v.reshape(-1, 1)
    w0x = W0[:, :3]
    w0p = W0[:, 3:]

    import functools
    body = functools.partial(_body, nb_total, cnt, B)

    vec_spec = lambda c: pl.BlockSpec((c, 1), lambda ph, b, nb: (0, 0))
    out = pl.pallas_call(
        body,
        grid=(2, B, nb_total),
        in_specs=[
            pl.BlockSpec((1, 3, blk), lambda ph, b, nb: (b, 0, nb)),
            pl.BlockSpec((1, D, blk), lambda ph, b, nb: (b, 0, nb)),
            pl.BlockSpec((C0, 3), lambda ph, b, nb: (0, 0)),
            pl.BlockSpec((C0, D), lambda ph, b, nb: (0, 0)),
            vec_spec(C0), vec_spec(C0), vec_spec(C0),
            pl.BlockSpec((C1, C0), lambda ph, b, nb: (0, 0)),
            vec_spec(C1), vec_spec(C1), vec_spec(C1),
        ],
        out_specs=pl.BlockSpec((C1, B), lambda ph, b, nb: (0, 0)),
        out_shape=jax.ShapeDtypeStruct((C1, B), jnp.float32),
        scratch_shapes=[
            pltpu.VMEM((C0, 1), jnp.float32),
            pltpu.VMEM((C0, 1), jnp.float32),
            pltpu.VMEM((C1, 1), jnp.float32),
            pltpu.VMEM((C1, 1), jnp.float32),
            pltpu.VMEM((C1, B), jnp.float32),
            pltpu.VMEM((C1, B), jnp.float32),
        ],
    )(xyz, points, w0x, w0p, col(b0), col(gamma0), col(beta0),
      W1, col(b1), col(gamma1), col(beta1))

    new_points = out.T.reshape(B, C1, 1)
    new_xyz = jnp.zeros((B, 3, 1), jnp.float32)
    return new_xyz, new_points
